# baseline (speedup 1.0000x reference)
import functools

import jax
import jax.numpy as jnp
from jax import lax
from jax.experimental import pallas as pl
from jax.experimental.pallas import tpu as pltpu
from jax.experimental.pallas import tpu_sc as plsc

_VOCAB_SIZE = 1000

_NC, _NS, _L = 2, 16, 16
_NW = _NC * _NS
_CH = 2


@functools.cache
def _build(nrows, ncols):
    assert ncols % (_NW * _CH * 128) == 0
    cols_w = ncols // _NW
    cw = cols_w // _CH
    vecs = cw // _L

    mesh = plsc.VectorSubcoreMesh(core_axis_name="c", subcore_axis_name="s")

    @functools.partial(
        pl.kernel,
        out_type=jax.ShapeDtypeStruct((nrows, ncols), jnp.int32),
        mesh=mesh,
        scratch_types=(
            [pltpu.VMEM((nrows, cw), jnp.int32) for _ in range(_CH)]
            + [pltpu.SemaphoreType.DMA for _ in range(2 * _CH)]
        ),
        compiler_params=pltpu.CompilerParams(use_tc_tiling_on_sc=True),
    )
    def body(x_hbm, out_hbm, *scratch):
        ibufs = scratch[:_CH]
        sin = scratch[_CH : 2 * _CH]
        sout = scratch[2 * _CH :]
        wid = lax.axis_index("s") * _NC + lax.axis_index("c")
        base = wid * cols_w

        h_in = [
            pltpu.async_copy(
                x_hbm.at[:, pl.ds(base + c * cw, cw)], ibufs[c], sin[c]
            )
            for c in range(_CH)
        ]
        h_out = []
        for c in range(_CH):
            h_in[c].wait()
            buf = ibufs[c]

            @plsc.parallel_loop(0, nrows, unroll=2)
            def _(r, buf=buf):
                for ci in range(vecs):
                    v = buf[r, pl.ds(ci * _L, _L)]
                    ok = plsc.bitcast(v, jnp.uint32) < _VOCAB_SIZE
                    buf[r, pl.ds(ci * _L, _L)] = jnp.where(ok, v, jnp.int32(-1))
            h_out.append(
                pltpu.async_copy(
                    buf, out_hbm.at[:, pl.ds(base + c * cw, cw)], sout[c]
                )
            )
        for h in h_out:
            h.wait()

    return body


def kernel(x):
    xt = x.T
    return _build(*xt.shape)(xt).T


# speedup vs baseline: 1.0190x; 1.0190x over previous
import functools

import jax
import jax.numpy as jnp
from jax import lax
from jax.experimental import pallas as pl
from jax.experimental.pallas import tpu as pltpu
from jax.experimental.pallas import tpu_sc as plsc

_VOCAB_SIZE = 1000

_NC, _NS, _L = 2, 16, 16
_NW = _NC * _NS
_CH = 4


@functools.cache
def _build(nrows, ncols):
    assert ncols % (_NW * _CH * 128) == 0
    cols_w = ncols // _NW
    cw = cols_w // _CH
    vecs = cw // _L

    mesh = plsc.VectorSubcoreMesh(core_axis_name="c", subcore_axis_name="s")

    @functools.partial(
        pl.kernel,
        out_type=jax.ShapeDtypeStruct((nrows, ncols), jnp.int32),
        mesh=mesh,
        scratch_types=(
            [pltpu.VMEM((nrows, cw), jnp.int32) for _ in range(_CH)]
            + [pltpu.SemaphoreType.DMA for _ in range(2 * _CH)]
        ),
        compiler_params=pltpu.CompilerParams(use_tc_tiling_on_sc=True),
    )
    def body(x_hbm, out_hbm, *scratch):
        ibufs = scratch[:_CH]
        sin = scratch[_CH : 2 * _CH]
        sout = scratch[2 * _CH :]
        wid = lax.axis_index("s") * _NC + lax.axis_index("c")
        base = wid * cols_w

        h_in = [
            pltpu.async_copy(
                x_hbm.at[:, pl.ds(base + c * cw, cw)], ibufs[c], sin[c]
            )
            for c in range(_CH)
        ]
        h_out = []
        for c in range(_CH):
            h_in[c].wait()
            buf = ibufs[c]

            @plsc.parallel_loop(0, nrows, unroll=2)
            def _(r, buf=buf):
                for ci in range(vecs):
                    v = buf[r, pl.ds(ci * _L, _L)]
                    ok = plsc.bitcast(v, jnp.uint32) < _VOCAB_SIZE
                    buf[r, pl.ds(ci * _L, _L)] = jnp.where(ok, v, jnp.int32(-1))
            h_out.append(
                pltpu.async_copy(
                    buf, out_hbm.at[:, pl.ds(base + c * cw, cw)], sout[c]
                )
            )
        for h in h_out:
            h.wait()

    return body


def kernel(x):
    xt = x.T
    return _build(*xt.shape)(xt).T
